# baseline (speedup 1.0000x reference)
import functools

import jax
import jax.numpy as jnp
from jax import lax
from jax.experimental import pallas as pl
from jax.experimental.pallas import tpu as pltpu
from jax.experimental.pallas import tpu_sc as plsc

NC = 2
NS = 16
NW = NC * NS
K = 128
NBUF = 3
LANES = 16


def _mesh():
  return plsc.VectorSubcoreMesh(core_axis_name="c", subcore_axis_name="s")


def _fill_rows(buf, nrows, ncols, value):
  v = jnp.full((LANES,), value, jnp.float32)

  def row(i, carry):
    for l in range(ncols // LANES):
      buf[i, pl.ds(l * LANES, LANES)] = v
    return carry

  lax.fori_loop(0, nrows, row, 0)


def _nblk(n):
  return (n + 1 + K - 1) // K


def _init_acc(zeros_v, acc, sid, nblk):
  for i in range(-(-nblk // NS)):
    blk = sid + i * NS
    @pl.when(blk < nblk)
    def _():
      pltpu.sync_copy(zeros_v, acc.at[pl.ds(blk * K, K)])


def _drain_acc(acc, out_hbm, cid, sid, nblk):
  for i in range(-(-nblk // NS)):
    blk = sid + i * NS
    @pl.when(blk < nblk)
    def _():
      pltpu.sync_copy(acc.at[pl.ds(blk * K, K)],
                      out_hbm.at[cid, pl.ds(blk * K, K)])


DEGW = 128


def _make_deg(n, nch):
  nblk = _nblk(n)
  acc_rows = nblk * K

  def body(dstv_hbm, out_hbm, dst_v, ones_v, acc):
    cid = lax.axis_index("c")
    sid = lax.axis_index("s")
    wid = sid * NC + cid
    pltpu.sync_copy(dstv_hbm.at[wid], dst_v)
    _fill_rows(ones_v, K, DEGW, 0.0)
    _init_acc(ones_v, acc, sid, nblk)
    _fill_rows(ones_v, K, DEGW, 1.0)
    plsc.subcore_barrier()

    def chunk(j, carry):
      pltpu.sync_copy(ones_v, acc.at[dst_v.at[j]], add=True)
      return carry

    lax.fori_loop(0, nch, chunk, 0)
    plsc.subcore_barrier()
    _drain_acc(acc, out_hbm, cid, sid, nblk)

  return pl.kernel(
      body,
      out_type=jax.ShapeDtypeStruct((NC, acc_rows, DEGW), jnp.float32),
      mesh=_mesh(),
      scratch_types=[
          pltpu.VMEM((nch, K), jnp.int32),
          pltpu.VMEM((K, DEGW), jnp.float32),
          pltpu.VMEM_SHARED((acc_rows, DEGW), jnp.float32),
      ],
  )


IDXBITS = 14


def _make_agg(n, d, c0, c1):
  pa = -(-(c0 // 2) // 8) * 8
  pb = c0 - pa
  assert pb >= 0 and pb % 2 == 0 and c1 % 2 == 0 and c1 <= pa
  assert n < (1 << IDXBITS)
  nblk = _nblk(n)
  acc_rows = nblk * K
  mask = (1 << IDXBITS) - 1

  def body(xs_hbm, pidx_hbm, out_hbm, pidx_v, idx_v, buf0, buf1, gsem0,
           gsem1, acc):
    bufs = [buf0, buf1]
    gsem = [gsem0, gsem1]
    cid = lax.axis_index("c")
    sid = lax.axis_index("s")
    wid = sid * NC + cid
    nch = lax.select(cid == 0, c0, c1)
    pltpu.sync_copy(pidx_hbm.at[wid, pl.ds(0, pa)], pidx_v)
    _fill_rows(buf0, K, d, 0.0)
    _init_acc(buf0, acc, sid, nblk)
    plsc.subcore_barrier()

    def unpack(j, b):
      def row(l, carry):
        w = pidx_v[j, pl.ds(l * LANES, LANES)]
        idx_v[b, 0, pl.ds(l * LANES, LANES)] = w & mask
        idx_v[b, 1, pl.ds(l * LANES, LANES)] = w >> IDXBITS
        return carry

      lax.fori_loop(0, K // LANES, row, 0)

    def fire(j, b):
      pltpu.async_copy(xs_hbm.at[idx_v.at[b, 0]], bufs[b], gsem[b])

    def wait_gather(b):
      pltpu.make_async_copy(xs_hbm.at[pl.ds(0, K)], bufs[b], gsem[b]).wait()

    def scatter(b):
      pltpu.sync_copy(bufs[b], acc.at[idx_v.at[b, 1]], add=True)

    def run_chunks(count):
      @pl.when(count > 0)
      def _():
        unpack(0, 0)
        fire(0, 0)

      def step(t, carry):
        j = 2 * t
        unpack(j + 1, 1)
        fire(j + 1, 1)
        wait_gather(0)
        scatter(0)

        @pl.when(j + 2 < count)
        def _():
          unpack(j + 2, 0)
          fire(j + 2, 0)

        wait_gather(1)
        scatter(1)
        return carry

      lax.fori_loop(0, count // 2, step, 0)

    n_a = lax.min(nch, pa)
    run_chunks(n_a)

    if pb > 0:
      @pl.when(nch > pa)
      def _():
        pltpu.sync_copy(pidx_hbm.at[wid, pl.ds(pa, pb)],
                        pidx_v.at[pl.ds(0, pb)])
        run_chunks(nch - pa)

    plsc.subcore_barrier()
    _drain_acc(acc, out_hbm, cid, sid, nblk)

  return pl.kernel(
      body,
      out_type=jax.ShapeDtypeStruct((NC, acc_rows, d), jnp.float32),
      mesh=_mesh(),
      scratch_types=[
          pltpu.VMEM((pa, K), jnp.int32),
          pltpu.VMEM((2, 2, K), jnp.int32),
          pltpu.VMEM((K, d), jnp.float32),
          pltpu.VMEM((K, d), jnp.float32),
          pltpu.SemaphoreType.DMA,
          pltpu.SemaphoreType.DMA,
          pltpu.VMEM_SHARED((acc_rows, d), jnp.float32),
      ],
  )


BN = 512


def _prep_body(x_ref, w_ref, degp_ref, xs_ref, dinv_ref):
  deg = degp_ref[0, :, 0:1] + degp_ref[1, :, 0:1] + 1.0
  dinv = lax.rsqrt(deg)
  xw = jnp.dot(x_ref[...], w_ref[...], preferred_element_type=jnp.float32)
  xs_ref[...] = dinv * xw
  dinv_ref[...] = dinv


def _mid_body(p_ref, xs_ref, dinv_ref, b1_ref, z_ref, u_ref):
  dinv = dinv_ref[...]
  agg = p_ref[0] + p_ref[1] + xs_ref[...]
  z = jnp.maximum(dinv * agg + b1_ref[...], 0.0)
  z_ref[...] = z
  u_ref[...] = dinv * z


def _fin_body(q_ref, u_ref, dinv_ref, w2_ref, b2_ref, out_ref):
  acc = q_ref[0] + q_ref[1] + u_ref[...]
  out_ref[...] = (dinv_ref[...]
                  * jnp.dot(acc, w2_ref[...],
                            preferred_element_type=jnp.float32)
                  + b2_ref[...])


def kernel(x, edge_index, W1, b1, W2, b2):
  n, in_dim = x.shape
  hid = W1.shape[1]
  out_dim = W2.shape[1]
  e = edge_index.shape[1]

  src = edge_index[0].astype(jnp.int32)
  dst = edge_index[1].astype(jnp.int32)

  nchd = -(-e // (NW * K))
  padd = NW * nchd * K - e
  dstv = jnp.concatenate([dst, jnp.full((padd,), n, jnp.int32)]).reshape(
      NW, nchd, K)
  degp = _make_deg(n, nchd)(dstv)

  need = -(-e // K)
  c1 = 2 * max(1, round(need * 0.15 / (2 * NS)))
  rem = -(-(need - NS * c1) // NS)
  c0 = -(-rem // 4) * 4
  e_pad = NS * (c0 + c1) * K
  pad = e_pad - e
  src_p = jnp.concatenate([src, jnp.zeros((pad,), jnp.int32)])
  dst_p = jnp.concatenate([dst, jnp.full((pad,), n, jnp.int32)])
  pidx_flat = (src_p | (dst_p << IDXBITS)).reshape(NS * (c0 + c1), K)
  p0 = pidx_flat[:NS * c0].reshape(NS, c0, K)
  p1 = pidx_flat[NS * c0:].reshape(NS, c1, K)
  p1 = jnp.pad(p1, ((0, 0), (0, c0 - c1), (0, 0)))
  pidx = jnp.stack([p0, p1], axis=1).reshape(NW, c0, K)

  grid = (pl.cdiv(n, BN),)
  xs, dinv = pl.pallas_call(
      _prep_body,
      grid=grid,
      in_specs=[
          pl.BlockSpec((BN, in_dim), lambda i: (i, 0)),
          pl.BlockSpec((in_dim, hid), lambda i: (0, 0)),
          pl.BlockSpec((NC, BN, DEGW), lambda i: (0, i, 0)),
      ],
      out_specs=[
          pl.BlockSpec((BN, hid), lambda i: (i, 0)),
          pl.BlockSpec((BN, 1), lambda i: (i, 0)),
      ],
      out_shape=[
          jax.ShapeDtypeStruct((n, hid), jnp.float32),
          jax.ShapeDtypeStruct((n, 1), jnp.float32),
      ],
  )(x, W1, degp)

  agg = _make_agg(n, hid, c0, c1)
  p = agg(xs, pidx)

  z, u = pl.pallas_call(
      _mid_body,
      grid=grid,
      in_specs=[
          pl.BlockSpec((NC, BN, hid), lambda i: (0, i, 0)),
          pl.BlockSpec((BN, hid), lambda i: (i, 0)),
          pl.BlockSpec((BN, 1), lambda i: (i, 0)),
          pl.BlockSpec((1, hid), lambda i: (0, 0)),
      ],
      out_specs=[
          pl.BlockSpec((BN, hid), lambda i: (i, 0)),
          pl.BlockSpec((BN, hid), lambda i: (i, 0)),
      ],
      out_shape=[
          jax.ShapeDtypeStruct((n, hid), jnp.float32),
          jax.ShapeDtypeStruct((n, hid), jnp.float32),
      ],
  )(p, xs, dinv, b1.reshape(1, hid))

  q = agg(u, pidx)

  logits = pl.pallas_call(
      _fin_body,
      grid=grid,
      in_specs=[
          pl.BlockSpec((NC, BN, hid), lambda i: (0, i, 0)),
          pl.BlockSpec((BN, hid), lambda i: (i, 0)),
          pl.BlockSpec((BN, 1), lambda i: (i, 0)),
          pl.BlockSpec((hid, out_dim), lambda i: (0, 0)),
          pl.BlockSpec((1, out_dim), lambda i: (0, 0)),
      ],
      out_specs=pl.BlockSpec((BN, out_dim), lambda i: (i, 0)),
      out_shape=jax.ShapeDtypeStruct((n, out_dim), jnp.float32),
  )(q, u, dinv, W2, b2.reshape(1, out_dim))

  return logits, z


# speedup vs baseline: 1.5069x; 1.5069x over previous
import functools

import jax
import jax.numpy as jnp
from jax import lax
from jax.experimental import pallas as pl
from jax.experimental.pallas import tpu as pltpu
from jax.experimental.pallas import tpu_sc as plsc

NC = 2
NS = 16
NW = NC * NS
K = 128
NBUF = 3
LANES = 16


def _mesh():
  return plsc.VectorSubcoreMesh(core_axis_name="c", subcore_axis_name="s")


def _fill_rows(buf, nrows, ncols, value):
  v = jnp.full((LANES,), value, jnp.float32)

  def row(i, carry):
    for l in range(ncols // LANES):
      buf[i, pl.ds(l * LANES, LANES)] = v
    return carry

  lax.fori_loop(0, nrows, row, 0)


def _nblk(n):
  return (n + 1 + K - 1) // K


def _init_acc(zeros_v, acc, sid, nblk):
  for i in range(-(-nblk // NS)):
    blk = sid + i * NS
    @pl.when(blk < nblk)
    def _():
      pltpu.sync_copy(zeros_v, acc.at[pl.ds(blk * K, K)])


def _drain_acc(acc, out_hbm, cid, sid, nblk):
  for i in range(-(-nblk // NS)):
    blk = sid + i * NS
    @pl.when(blk < nblk)
    def _():
      pltpu.sync_copy(acc.at[pl.ds(blk * K, K)],
                      out_hbm.at[cid, pl.ds(blk * K, K)])


DEGW = 128


def _make_deg(n, nch):
  nblk = _nblk(n)
  acc_rows = nblk * K

  def body(dstv_hbm, out_hbm, dst_v, ones_v, acc):
    cid = lax.axis_index("c")
    sid = lax.axis_index("s")
    wid = sid * NC + cid
    pltpu.sync_copy(dstv_hbm.at[wid], dst_v)
    _fill_rows(ones_v, K, DEGW, 0.0)
    _init_acc(ones_v, acc, sid, nblk)
    _fill_rows(ones_v, K, DEGW, 1.0)
    plsc.subcore_barrier()

    def chunk(j, carry):
      pltpu.sync_copy(ones_v, acc.at[dst_v.at[j]], add=True)
      return carry

    lax.fori_loop(0, nch, chunk, 0)
    plsc.subcore_barrier()
    _drain_acc(acc, out_hbm, cid, sid, nblk)

  return pl.kernel(
      body,
      out_type=jax.ShapeDtypeStruct((NC, acc_rows, DEGW), jnp.float32),
      mesh=_mesh(),
      scratch_types=[
          pltpu.VMEM((nch, K), jnp.int32),
          pltpu.VMEM((K, DEGW), jnp.float32),
          pltpu.VMEM_SHARED((acc_rows, DEGW), jnp.float32),
      ],
  )


IDXBITS = 14


def _make_agg(n, d, c0, c1):
  pa = -(-(c0 // 2) // 8) * 8
  pb = c0 - pa
  assert pb >= 0 and pb % 2 == 0 and c1 % 2 == 0 and c1 <= pa
  assert n < (1 << IDXBITS)
  nblk = _nblk(n)
  acc_rows = nblk * K
  mask = (1 << IDXBITS) - 1

  def body(xs_hbm, pidx_hbm, out_hbm, pidx_v, idx_v, buf0, buf1, gsem0,
           gsem1, acc):
    bufs = [buf0, buf1]
    gsem = [gsem0, gsem1]
    cid = lax.axis_index("c")
    sid = lax.axis_index("s")
    wid = sid * NC + cid
    nch = lax.select(cid == 0, c0, c1)
    pltpu.sync_copy(pidx_hbm.at[wid, pl.ds(0, pa)], pidx_v)
    _fill_rows(buf0, K, d, 0.0)
    _init_acc(buf0, acc, sid, nblk)
    plsc.subcore_barrier()

    def unpack(j, b):
      def row(l, carry):
        w = pidx_v[j, pl.ds(l * LANES, LANES)]
        idx_v[b, 0, pl.ds(l * LANES, LANES)] = w & mask
        idx_v[b, 1, pl.ds(l * LANES, LANES)] = w >> IDXBITS
        return carry

      lax.fori_loop(0, K // LANES, row, 0)

    def fire(j, b):
      pltpu.async_copy(xs_hbm.at[idx_v.at[b, 0]], bufs[b], gsem[b])

    def wait_gather(b):
      pltpu.make_async_copy(xs_hbm.at[pl.ds(0, K)], bufs[b], gsem[b]).wait()

    def scatter(b):
      pltpu.sync_copy(bufs[b], acc.at[idx_v.at[b, 1]], add=True)

    def run_chunks(count):
      @pl.when(count > 0)
      def _():
        unpack(0, 0)
        fire(0, 0)

      def step(t, carry):
        j = 2 * t
        unpack(j + 1, 1)
        fire(j + 1, 1)
        wait_gather(0)
        scatter(0)

        @pl.when(j + 2 < count)
        def _():
          unpack(j + 2, 0)
          fire(j + 2, 0)

        wait_gather(1)
        scatter(1)
        return carry

      lax.fori_loop(0, count // 2, step, 0)

    n_a = lax.min(nch, pa)
    run_chunks(n_a)

    if pb > 0:
      @pl.when(nch > pa)
      def _():
        pltpu.sync_copy(pidx_hbm.at[wid, pl.ds(pa, pb)],
                        pidx_v.at[pl.ds(0, pb)])
        run_chunks(nch - pa)

    plsc.subcore_barrier()
    _drain_acc(acc, out_hbm, cid, sid, nblk)

  return pl.kernel(
      body,
      out_type=jax.ShapeDtypeStruct((NC, acc_rows, d), jnp.float32),
      mesh=_mesh(),
      scratch_types=[
          pltpu.VMEM((pa, K), jnp.int32),
          pltpu.VMEM((2, 2, K), jnp.int32),
          pltpu.VMEM((K, d), jnp.float32),
          pltpu.VMEM((K, d), jnp.float32),
          pltpu.SemaphoreType.DMA,
          pltpu.SemaphoreType.DMA,
          pltpu.VMEM_SHARED((acc_rows, d), jnp.float32),
      ],
  )


BN = 512


def _prep_body(x_ref, w_ref, degp_ref, xs_ref, dinv_ref):
  deg = degp_ref[0, :, 0:1] + degp_ref[1, :, 0:1] + 1.0
  dinv = lax.rsqrt(deg)
  xw = jnp.dot(x_ref[...], w_ref[...], preferred_element_type=jnp.float32)
  xs_ref[...] = dinv * xw
  dinv_ref[...] = dinv


def _mid_body(p_ref, xs_ref, dinv_ref, b1_ref, z_ref, u_ref):
  dinv = dinv_ref[...]
  agg = p_ref[0] + p_ref[1] + xs_ref[...]
  z = jnp.maximum(dinv * agg + b1_ref[...], 0.0)
  z_ref[...] = z
  u_ref[...] = dinv * z


def _fin_body(q_ref, u_ref, dinv_ref, w2_ref, b2_ref, out_ref):
  acc = q_ref[0] + q_ref[1] + u_ref[...]
  out_ref[...] = (dinv_ref[...]
                  * jnp.dot(acc, w2_ref[...],
                            preferred_element_type=jnp.float32)
                  + b2_ref[...])


def kernel(x, edge_index, W1, b1, W2, b2):
  n, in_dim = x.shape
  hid = W1.shape[1]
  out_dim = W2.shape[1]
  e = edge_index.shape[1]

  src = edge_index[0].astype(jnp.int32)
  dst = edge_index[1].astype(jnp.int32)

  nchd = -(-e // (NW * K))
  padd = NW * nchd * K - e
  dstv = jnp.concatenate([dst, jnp.full((padd,), n, jnp.int32)]).reshape(
      NW, nchd, K)
  degp = _make_deg(n, nchd)(dstv)

  need = -(-e // K)
  c1 = 2 * max(1, round(need * 0.19 / (2 * NS)))
  rem = -(-(need - NS * c1) // NS)
  c0 = -(-rem // 4) * 4
  e_pad = NS * (c0 + c1) * K
  pad = e_pad - e
  src_p = jnp.concatenate([src, jnp.zeros((pad,), jnp.int32)])
  dst_p = jnp.concatenate([dst, jnp.full((pad,), n, jnp.int32)])
  pidx_flat = (src_p | (dst_p << IDXBITS)).reshape(NS * (c0 + c1), K)
  p0 = pidx_flat[:NS * c0].reshape(NS, c0, K)
  p1 = pidx_flat[NS * c0:].reshape(NS, c1, K)
  p1 = jnp.pad(p1, ((0, 0), (0, c0 - c1), (0, 0)))
  pidx = jnp.stack([p0, p1], axis=1).reshape(NW, c0, K)

  grid = (pl.cdiv(n, BN),)
  xs, dinv = pl.pallas_call(
      _prep_body,
      grid=grid,
      in_specs=[
          pl.BlockSpec((BN, in_dim), lambda i: (i, 0)),
          pl.BlockSpec((in_dim, hid), lambda i: (0, 0)),
          pl.BlockSpec((NC, BN, DEGW), lambda i: (0, i, 0)),
      ],
      out_specs=[
          pl.BlockSpec((BN, hid), lambda i: (i, 0)),
          pl.BlockSpec((BN, 1), lambda i: (i, 0)),
      ],
      out_shape=[
          jax.ShapeDtypeStruct((n, hid), jnp.float32),
          jax.ShapeDtypeStruct((n, 1), jnp.float32),
      ],
  )(x, W1, degp)

  agg = _make_agg(n, hid, c0, c1)
  p = agg(xs, pidx)

  z, u = pl.pallas_call(
      _mid_body,
      grid=grid,
      in_specs=[
          pl.BlockSpec((NC, BN, hid), lambda i: (0, i, 0)),
          pl.BlockSpec((BN, hid), lambda i: (i, 0)),
          pl.BlockSpec((BN, 1), lambda i: (i, 0)),
          pl.BlockSpec((1, hid), lambda i: (0, 0)),
      ],
      out_specs=[
          pl.BlockSpec((BN, hid), lambda i: (i, 0)),
          pl.BlockSpec((BN, hid), lambda i: (i, 0)),
      ],
      out_shape=[
          jax.ShapeDtypeStruct((n, hid), jnp.float32),
          jax.ShapeDtypeStruct((n, hid), jnp.float32),
      ],
  )(p, xs, dinv, b1.reshape(1, hid))

  q = agg(u, pidx)

  logits = pl.pallas_call(
      _fin_body,
      grid=grid,
      in_specs=[
          pl.BlockSpec((NC, BN, hid), lambda i: (0, i, 0)),
          pl.BlockSpec((BN, hid), lambda i: (i, 0)),
          pl.BlockSpec((BN, 1), lambda i: (i, 0)),
          pl.BlockSpec((hid, out_dim), lambda i: (0, 0)),
          pl.BlockSpec((1, out_dim), lambda i: (0, 0)),
      ],
      out_specs=pl.BlockSpec((BN, out_dim), lambda i: (i, 0)),
      out_shape=jax.ShapeDtypeStruct((n, out_dim), jnp.float32),
  )(q, u, dinv, W2, b2.reshape(1, out_dim))

  return logits, z


# speedup vs baseline: 1.5241x; 1.0114x over previous
import functools

import jax
import jax.numpy as jnp
from jax import lax
from jax.experimental import pallas as pl
from jax.experimental.pallas import tpu as pltpu
from jax.experimental.pallas import tpu_sc as plsc

NC = 2
NS = 16
NW = NC * NS
K = 128
NBUF = 3
LANES = 16


def _mesh():
  return plsc.VectorSubcoreMesh(core_axis_name="c", subcore_axis_name="s")


def _fill_rows(buf, nrows, ncols, value):
  v = jnp.full((LANES,), value, jnp.float32)

  def row(i, carry):
    for l in range(ncols // LANES):
      buf[i, pl.ds(l * LANES, LANES)] = v
    return carry

  lax.fori_loop(0, nrows, row, 0)


def _nblk(n):
  return (n + 1 + K - 1) // K


def _init_acc(zeros_v, acc, sid, nblk):
  for i in range(-(-nblk // NS)):
    blk = sid + i * NS
    @pl.when(blk < nblk)
    def _():
      pltpu.sync_copy(zeros_v, acc.at[pl.ds(blk * K, K)])


def _drain_acc(acc, out_hbm, cid, sid, nblk):
  for i in range(-(-nblk // NS)):
    blk = sid + i * NS
    @pl.when(blk < nblk)
    def _():
      pltpu.sync_copy(acc.at[pl.ds(blk * K, K)],
                      out_hbm.at[cid, pl.ds(blk * K, K)])


DEGW = 128


def _make_deg(n, nch):
  nblk = _nblk(n)
  acc_rows = nblk * K

  def body(dstv_hbm, out_hbm, dst_v, ones_v, acc):
    cid = lax.axis_index("c")
    sid = lax.axis_index("s")
    wid = sid * NC + cid
    pltpu.sync_copy(dstv_hbm.at[wid], dst_v)
    _fill_rows(ones_v, K, DEGW, 0.0)
    _init_acc(ones_v, acc, sid, nblk)
    _fill_rows(ones_v, K, DEGW, 1.0)
    plsc.subcore_barrier()

    def chunk(j, carry):
      pltpu.sync_copy(ones_v, acc.at[dst_v.at[j]], add=True)
      return carry

    lax.fori_loop(0, nch, chunk, 0)
    plsc.subcore_barrier()
    _drain_acc(acc, out_hbm, cid, sid, nblk)

  return pl.kernel(
      body,
      out_type=jax.ShapeDtypeStruct((NC, acc_rows, DEGW), jnp.float32),
      mesh=_mesh(),
      scratch_types=[
          pltpu.VMEM((nch, K), jnp.int32),
          pltpu.VMEM((K, DEGW), jnp.float32),
          pltpu.VMEM_SHARED((acc_rows, DEGW), jnp.float32),
      ],
  )


IDXBITS = 14


def _make_agg(n, d, c0, c1):
  pa = -(-(c0 // 2) // 8) * 8
  pb = c0 - pa
  assert pb >= 0 and pb % 2 == 0 and c1 % 2 == 0 and c1 <= pa
  assert n < (1 << IDXBITS)
  nblk = _nblk(n)
  acc_rows = nblk * K
  mask = (1 << IDXBITS) - 1

  def body(xs_hbm, pidx_hbm, out_hbm, pidx_v, idx_v, buf0, buf1, gsem0,
           gsem1, acc):
    bufs = [buf0, buf1]
    gsem = [gsem0, gsem1]
    cid = lax.axis_index("c")
    sid = lax.axis_index("s")
    wid = sid * NC + cid
    nch = lax.select(cid == 0, c0, c1)
    pltpu.sync_copy(pidx_hbm.at[wid, pl.ds(0, pa)], pidx_v)
    _fill_rows(buf0, K, d, 0.0)
    _init_acc(buf0, acc, sid, nblk)
    plsc.subcore_barrier()

    def unpack(j, b):
      def row(l, carry):
        w = pidx_v[j, pl.ds(l * LANES, LANES)]
        idx_v[b, 0, pl.ds(l * LANES, LANES)] = w & mask
        idx_v[b, 1, pl.ds(l * LANES, LANES)] = w >> IDXBITS
        return carry

      lax.fori_loop(0, K // LANES, row, 0)

    def fire(j, b):
      pltpu.async_copy(xs_hbm.at[idx_v.at[b, 0]], bufs[b], gsem[b])

    def wait_gather(b):
      pltpu.make_async_copy(xs_hbm.at[pl.ds(0, K)], bufs[b], gsem[b]).wait()

    def scatter(b):
      pltpu.sync_copy(bufs[b], acc.at[idx_v.at[b, 1]], add=True)

    def run_chunks(count):
      @pl.when(count > 0)
      def _():
        unpack(0, 0)
        fire(0, 0)

      def step(t, carry):
        j = 2 * t
        unpack(j + 1, 1)
        fire(j + 1, 1)
        wait_gather(0)
        scatter(0)

        @pl.when(j + 2 < count)
        def _():
          unpack(j + 2, 0)
          fire(j + 2, 0)

        wait_gather(1)
        scatter(1)
        return carry

      lax.fori_loop(0, count // 2, step, 0)

    n_a = lax.min(nch, pa)
    run_chunks(n_a)

    if pb > 0:
      @pl.when(nch > pa)
      def _():
        pltpu.sync_copy(pidx_hbm.at[wid, pl.ds(pa, pb)],
                        pidx_v.at[pl.ds(0, pb)])
        run_chunks(nch - pa)

    plsc.subcore_barrier()
    _drain_acc(acc, out_hbm, cid, sid, nblk)

  return pl.kernel(
      body,
      out_type=jax.ShapeDtypeStruct((NC, acc_rows, d), jnp.float32),
      mesh=_mesh(),
      scratch_types=[
          pltpu.VMEM((pa, K), jnp.int32),
          pltpu.VMEM((2, 2, K), jnp.int32),
          pltpu.VMEM((K, d), jnp.float32),
          pltpu.VMEM((K, d), jnp.float32),
          pltpu.SemaphoreType.DMA,
          pltpu.SemaphoreType.DMA,
          pltpu.VMEM_SHARED((acc_rows, d), jnp.float32),
      ],
  )


BN = 512


def _prep_body(x_ref, w_ref, degp_ref, xs_ref, dinv_ref):
  deg = degp_ref[0, :, 0:1] + degp_ref[1, :, 0:1] + 1.0
  dinv = lax.rsqrt(deg)
  xw = jnp.dot(x_ref[...], w_ref[...], preferred_element_type=jnp.float32)
  xs_ref[...] = dinv * xw
  dinv_ref[...] = dinv


def _mid_body(p_ref, xs_ref, dinv_ref, b1_ref, z_ref, u_ref):
  dinv = dinv_ref[...]
  agg = p_ref[0] + p_ref[1] + xs_ref[...]
  z = jnp.maximum(dinv * agg + b1_ref[...], 0.0)
  z_ref[...] = z
  u_ref[...] = dinv * z


def _fin_body(q_ref, u_ref, dinv_ref, w2_ref, b2_ref, out_ref):
  acc = q_ref[0] + q_ref[1] + u_ref[...]
  out_ref[...] = (dinv_ref[...]
                  * jnp.dot(acc, w2_ref[...],
                            preferred_element_type=jnp.float32)
                  + b2_ref[...])


def kernel(x, edge_index, W1, b1, W2, b2):
  n, in_dim = x.shape
  hid = W1.shape[1]
  out_dim = W2.shape[1]
  e = edge_index.shape[1]

  src = edge_index[0].astype(jnp.int32)
  dst = edge_index[1].astype(jnp.int32)

  nchd = -(-e // (NW * K))
  padd = NW * nchd * K - e
  dstv = jnp.concatenate([dst, jnp.full((padd,), n, jnp.int32)]).reshape(
      NW, nchd, K)
  degp = _make_deg(n, nchd)(dstv)

  need = -(-e // K)
  c1 = 2 * max(1, round(need * 0.167 / (2 * NS)))
  rem = -(-(need - NS * c1) // NS)
  c0 = -(-rem // 4) * 4
  e_pad = NS * (c0 + c1) * K
  pad = e_pad - e
  src_p = jnp.concatenate([src, jnp.zeros((pad,), jnp.int32)])
  dst_p = jnp.concatenate([dst, jnp.full((pad,), n, jnp.int32)])
  pidx_flat = (src_p | (dst_p << IDXBITS)).reshape(NS * (c0 + c1), K)
  p0 = pidx_flat[:NS * c0].reshape(NS, c0, K)
  p1 = pidx_flat[NS * c0:].reshape(NS, c1, K)
  p1 = jnp.pad(p1, ((0, 0), (0, c0 - c1), (0, 0)))
  pidx = jnp.stack([p0, p1], axis=1).reshape(NW, c0, K)

  grid = (pl.cdiv(n, BN),)
  xs, dinv = pl.pallas_call(
      _prep_body,
      grid=grid,
      in_specs=[
          pl.BlockSpec((BN, in_dim), lambda i: (i, 0)),
          pl.BlockSpec((in_dim, hid), lambda i: (0, 0)),
          pl.BlockSpec((NC, BN, DEGW), lambda i: (0, i, 0)),
      ],
      out_specs=[
          pl.BlockSpec((BN, hid), lambda i: (i, 0)),
          pl.BlockSpec((BN, 1), lambda i: (i, 0)),
      ],
      out_shape=[
          jax.ShapeDtypeStruct((n, hid), jnp.float32),
          jax.ShapeDtypeStruct((n, 1), jnp.float32),
      ],
  )(x, W1, degp)

  agg = _make_agg(n, hid, c0, c1)
  p = agg(xs, pidx)

  z, u = pl.pallas_call(
      _mid_body,
      grid=grid,
      in_specs=[
          pl.BlockSpec((NC, BN, hid), lambda i: (0, i, 0)),
          pl.BlockSpec((BN, hid), lambda i: (i, 0)),
          pl.BlockSpec((BN, 1), lambda i: (i, 0)),
          pl.BlockSpec((1, hid), lambda i: (0, 0)),
      ],
      out_specs=[
          pl.BlockSpec((BN, hid), lambda i: (i, 0)),
          pl.BlockSpec((BN, hid), lambda i: (i, 0)),
      ],
      out_shape=[
          jax.ShapeDtypeStruct((n, hid), jnp.float32),
          jax.ShapeDtypeStruct((n, hid), jnp.float32),
      ],
  )(p, xs, dinv, b1.reshape(1, hid))

  q = agg(u, pidx)

  logits = pl.pallas_call(
      _fin_body,
      grid=grid,
      in_specs=[
          pl.BlockSpec((NC, BN, hid), lambda i: (0, i, 0)),
          pl.BlockSpec((BN, hid), lambda i: (i, 0)),
          pl.BlockSpec((BN, 1), lambda i: (i, 0)),
          pl.BlockSpec((hid, out_dim), lambda i: (0, 0)),
          pl.BlockSpec((1, out_dim), lambda i: (0, 0)),
      ],
      out_specs=pl.BlockSpec((BN, out_dim), lambda i: (i, 0)),
      out_shape=jax.ShapeDtypeStruct((n, out_dim), jnp.float32),
  )(q, u, dinv, W2, b2.reshape(1, out_dim))

  return logits, z


# speedup vs baseline: 1.5254x; 1.0008x over previous
import jax
import jax.numpy as jnp
from jax import lax
from jax.experimental import pallas as pl
from jax.experimental.pallas import tpu as pltpu
from jax.experimental.pallas import tpu_sc as plsc

NC = 2
NS = 16
NW = NC * NS
K = 128
LANES = 16


def _mesh():
  return plsc.VectorSubcoreMesh(core_axis_name="c", subcore_axis_name="s")


def _fill_rows(buf, nrows, ncols, value):
  v = jnp.full((LANES,), value, jnp.float32)

  def row(i, carry):
    for l in range(ncols // LANES):
      buf[i, pl.ds(l * LANES, LANES)] = v
    return carry

  lax.fori_loop(0, nrows, row, 0)


def _nblk(n):
  return (n + 1 + K - 1) // K


def _init_acc(zeros_v, acc, sid, nblk):
  for i in range(-(-nblk // NS)):
    blk = sid + i * NS
    @pl.when(blk < nblk)
    def _():
      pltpu.sync_copy(zeros_v, acc.at[pl.ds(blk * K, K)])


def _drain_acc(acc, out_hbm, cid, sid, nblk):
  for i in range(-(-nblk // NS)):
    blk = sid + i * NS
    @pl.when(blk < nblk)
    def _():
      pltpu.sync_copy(acc.at[pl.ds(blk * K, K)],
                      out_hbm.at[cid, pl.ds(blk * K, K)])


DEGW = 128


def _make_deg(n, nch):
  nblk = _nblk(n)
  acc_rows = nblk * K

  def body(dstv_hbm, out_hbm, dst_v, ones_v, acc):
    cid = lax.axis_index("c")
    sid = lax.axis_index("s")
    wid = sid * NC + cid
    pltpu.sync_copy(dstv_hbm.at[wid], dst_v)
    _fill_rows(ones_v, K, DEGW, 0.0)
    _init_acc(ones_v, acc, sid, nblk)
    _fill_rows(ones_v, K, DEGW, 1.0)
    plsc.subcore_barrier()

    def chunk(j, carry):
      pltpu.sync_copy(ones_v, acc.at[dst_v.at[j]], add=True)
      return carry

    lax.fori_loop(0, nch, chunk, 0)
    plsc.subcore_barrier()
    _drain_acc(acc, out_hbm, cid, sid, nblk)

  return pl.kernel(
      body,
      out_type=jax.ShapeDtypeStruct((NC, acc_rows, DEGW), jnp.float32),
      mesh=_mesh(),
      scratch_types=[
          pltpu.VMEM((nch, K), jnp.int32),
          pltpu.VMEM((K, DEGW), jnp.float32),
          pltpu.VMEM_SHARED((acc_rows, DEGW), jnp.float32),
      ],
  )


IDXBITS = 14


def _make_agg(n, d, c0, c1):
  pa = -(-(c0 // 2) // 8) * 8
  pb = c0 - pa
  assert pb >= 0 and pb % 2 == 0 and c1 % 2 == 0 and c1 <= pa
  assert n < (1 << IDXBITS)
  nblk = _nblk(n)
  acc_rows = nblk * K
  mask = (1 << IDXBITS) - 1

  def body(xs_hbm, pidx_hbm, out_hbm, pidx_v, idx_v, buf0, buf1, gsem0,
           gsem1, acc):
    bufs = [buf0, buf1]
    gsem = [gsem0, gsem1]
    cid = lax.axis_index("c")
    sid = lax.axis_index("s")
    wid = sid * NC + cid
    nch = lax.select(cid == 0, c0, c1)
    pltpu.sync_copy(pidx_hbm.at[wid, pl.ds(0, pa)], pidx_v)
    _fill_rows(buf0, K, d, 0.0)
    _init_acc(buf0, acc, sid, nblk)
    plsc.subcore_barrier()

    def unpack(j, b):
      def row(l, carry):
        w = pidx_v[j, pl.ds(l * LANES, LANES)]
        idx_v[b, 0, pl.ds(l * LANES, LANES)] = w & mask
        idx_v[b, 1, pl.ds(l * LANES, LANES)] = w >> IDXBITS
        return carry

      lax.fori_loop(0, K // LANES, row, 0)

    def fire(j, b):
      pltpu.async_copy(xs_hbm.at[idx_v.at[b, 0]], bufs[b], gsem[b])

    def wait_gather(b):
      pltpu.make_async_copy(xs_hbm.at[pl.ds(0, K)], bufs[b], gsem[b]).wait()

    def scatter(b):
      pltpu.sync_copy(bufs[b], acc.at[idx_v.at[b, 1]], add=True)

    def run_chunks(count):
      @pl.when(count > 0)
      def _():
        unpack(0, 0)
        fire(0, 0)

      def step(t, carry):
        j = 2 * t
        unpack(j + 1, 1)
        fire(j + 1, 1)
        wait_gather(0)
        scatter(0)

        @pl.when(j + 2 < count)
        def _():
          unpack(j + 2, 0)
          fire(j + 2, 0)

        wait_gather(1)
        scatter(1)
        return carry

      lax.fori_loop(0, count // 2, step, 0)

    n_a = lax.min(nch, pa)
    run_chunks(n_a)

    if pb > 0:
      @pl.when(nch > pa)
      def _():
        pltpu.sync_copy(pidx_hbm.at[wid, pl.ds(pa, pb)],
                        pidx_v.at[pl.ds(0, pb)])
        run_chunks(nch - pa)

    plsc.subcore_barrier()
    _drain_acc(acc, out_hbm, cid, sid, nblk)

  return pl.kernel(
      body,
      out_type=jax.ShapeDtypeStruct((NC, acc_rows, d), jnp.float32),
      mesh=_mesh(),
      scratch_types=[
          pltpu.VMEM((pa, K), jnp.int32),
          pltpu.VMEM((2, 2, K), jnp.int32),
          pltpu.VMEM((K, d), jnp.float32),
          pltpu.VMEM((K, d), jnp.float32),
          pltpu.SemaphoreType.DMA,
          pltpu.SemaphoreType.DMA,
          pltpu.VMEM_SHARED((acc_rows, d), jnp.float32),
      ],
  )


BN = 512


def _prep_body(x_ref, w_ref, degp_ref, xs_ref, dinv_ref):
  deg = degp_ref[0, :, 0:1] + degp_ref[1, :, 0:1] + 1.0
  dinv = lax.rsqrt(deg)
  xw = jnp.dot(x_ref[...], w_ref[...], preferred_element_type=jnp.float32)
  xs_ref[...] = dinv * xw
  dinv_ref[...] = dinv


def _mid_body(p_ref, xs_ref, dinv_ref, b1_ref, z_ref, u_ref):
  dinv = dinv_ref[...]
  agg = p_ref[0] + p_ref[1] + xs_ref[...]
  z = jnp.maximum(dinv * agg + b1_ref[...], 0.0)
  z_ref[...] = z
  u_ref[...] = dinv * z


def _fin_body(q_ref, u_ref, dinv_ref, w2_ref, b2_ref, out_ref):
  acc = q_ref[0] + q_ref[1] + u_ref[...]
  out_ref[...] = (dinv_ref[...]
                  * jnp.dot(acc, w2_ref[...],
                            preferred_element_type=jnp.float32)
                  + b2_ref[...])


def kernel(x, edge_index, W1, b1, W2, b2):
  n, in_dim = x.shape
  hid = W1.shape[1]
  out_dim = W2.shape[1]
  e = edge_index.shape[1]

  src = edge_index[0].astype(jnp.int32)
  dst = edge_index[1].astype(jnp.int32)

  nchd = -(-e // (NW * K))
  padd = NW * nchd * K - e
  dstv = jnp.concatenate([dst, jnp.full((padd,), n, jnp.int32)]).reshape(
      NW, nchd, K)
  degp = _make_deg(n, nchd)(dstv)

  need = -(-e // K)
  c1 = 2 * max(1, round(need * 0.167 / (2 * NS)))
  rem = -(-(need - NS * c1) // NS)
  c0 = -(-rem // 4) * 4
  e_pad = NS * (c0 + c1) * K
  pad = e_pad - e
  src_p = jnp.concatenate([src, jnp.zeros((pad,), jnp.int32)])
  dst_p = jnp.concatenate([dst, jnp.full((pad,), n, jnp.int32)])
  pidx_flat = (src_p | (dst_p << IDXBITS)).reshape(NS * (c0 + c1), K)
  p0 = pidx_flat[:NS * c0].reshape(NS, c0, K)
  p1 = pidx_flat[NS * c0:].reshape(NS, c1, K)
  p1 = jnp.pad(p1, ((0, 0), (0, c0 - c1), (0, 0)))
  pidx = jnp.stack([p0, p1], axis=1).reshape(NW, c0, K)

  grid = (pl.cdiv(n, BN),)
  xs, dinv = pl.pallas_call(
      _prep_body,
      grid=grid,
      in_specs=[
          pl.BlockSpec((BN, in_dim), lambda i: (i, 0)),
          pl.BlockSpec((in_dim, hid), lambda i: (0, 0)),
          pl.BlockSpec((NC, BN, DEGW), lambda i: (0, i, 0)),
      ],
      out_specs=[
          pl.BlockSpec((BN, hid), lambda i: (i, 0)),
          pl.BlockSpec((BN, 1), lambda i: (i, 0)),
      ],
      out_shape=[
          jax.ShapeDtypeStruct((n, hid), jnp.float32),
          jax.ShapeDtypeStruct((n, 1), jnp.float32),
      ],
  )(x, W1, degp)

  agg = _make_agg(n, hid, c0, c1)
  p = agg(xs, pidx)

  z, u = pl.pallas_call(
      _mid_body,
      grid=grid,
      in_specs=[
          pl.BlockSpec((NC, BN, hid), lambda i: (0, i, 0)),
          pl.BlockSpec((BN, hid), lambda i: (i, 0)),
          pl.BlockSpec((BN, 1), lambda i: (i, 0)),
          pl.BlockSpec((1, hid), lambda i: (0, 0)),
      ],
      out_specs=[
          pl.BlockSpec((BN, hid), lambda i: (i, 0)),
          pl.BlockSpec((BN, hid), lambda i: (i, 0)),
      ],
      out_shape=[
          jax.ShapeDtypeStruct((n, hid), jnp.float32),
          jax.ShapeDtypeStruct((n, hid), jnp.float32),
      ],
  )(p, xs, dinv, b1.reshape(1, hid))

  q = agg(u, pidx)

  logits = pl.pallas_call(
      _fin_body,
      grid=grid,
      in_specs=[
          pl.BlockSpec((NC, BN, hid), lambda i: (0, i, 0)),
          pl.BlockSpec((BN, hid), lambda i: (i, 0)),
          pl.BlockSpec((BN, 1), lambda i: (i, 0)),
          pl.BlockSpec((hid, out_dim), lambda i: (0, 0)),
          pl.BlockSpec((1, out_dim), lambda i: (0, 0)),
      ],
      out_specs=pl.BlockSpec((BN, out_dim), lambda i: (i, 0)),
      out_shape=jax.ShapeDtypeStruct((n, out_dim), jnp.float32),
  )(q, u, dinv, W2, b2.reshape(1, out_dim))

  return logits, z
